# Initial kernel scaffold; baseline (speedup 1.0000x reference)
import jax
import jax.numpy as jnp
from jax.experimental import pallas as pl


def kernel(x, expert_embeddings, bias):
    raise NotImplementedError("write your pallas kernel here")


# speedup vs baseline: 1.3674x; 1.3674x over previous
import functools

import jax
import jax.numpy as jnp
from jax.experimental import pallas as pl

_TOPK = 8
_NUM_EXPERTS = 64
_BLOCK_ROWS = 512


def _gate_kernel(x_ref, e_ref, b_ref, w_ref, i_ref, s_ref):
    logits = jax.lax.dot_general(
        x_ref[...], e_ref[...],
        dimension_numbers=(((1,), (1,)), ((), ())),
        preferred_element_type=jnp.float32,
    )
    scores = jax.nn.sigmoid(logits) + b_ref[...]
    s_ref[...] = scores

    iota = jax.lax.broadcasted_iota(jnp.int32, scores.shape, 1)
    vals = scores
    neg_inf = jnp.float32(-jnp.inf)
    top_v = []
    top_i = []
    for _ in range(_TOPK):
        m = jnp.max(vals, axis=-1, keepdims=True)
        idx = jnp.min(jnp.where(vals == m, iota, _NUM_EXPERTS),
                      axis=-1, keepdims=True)
        top_v.append(m)
        top_i.append(idx)
        vals = jnp.where(iota == idx, neg_inf, vals)
    v = jnp.concatenate(top_v, axis=-1)
    i_ref[...] = jnp.concatenate(top_i, axis=-1)
    w_ref[...] = v / jnp.sum(v, axis=-1, keepdims=True)


@jax.jit
def kernel(x, expert_embeddings, bias):
    n_rows, _ = x.shape
    n_exp = expert_embeddings.shape[0]
    grid = (n_rows // _BLOCK_ROWS,)
    bias2d = bias.reshape(1, n_exp)
    weights, indices, scores = pl.pallas_call(
        _gate_kernel,
        grid=grid,
        in_specs=[
            pl.BlockSpec((_BLOCK_ROWS, x.shape[1]), lambda i: (i, 0)),
            pl.BlockSpec((n_exp, x.shape[1]), lambda i: (0, 0)),
            pl.BlockSpec((1, n_exp), lambda i: (0, 0)),
        ],
        out_specs=[
            pl.BlockSpec((_BLOCK_ROWS, _TOPK), lambda i: (i, 0)),
            pl.BlockSpec((_BLOCK_ROWS, _TOPK), lambda i: (i, 0)),
            pl.BlockSpec((_BLOCK_ROWS, n_exp), lambda i: (i, 0)),
        ],
        out_shape=[
            jax.ShapeDtypeStruct((n_rows, _TOPK), jnp.float32),
            jax.ShapeDtypeStruct((n_rows, _TOPK), jnp.int32),
            jax.ShapeDtypeStruct((n_rows, n_exp), jnp.float32),
        ],
    )(x, expert_embeddings, bias2d)
    return weights.astype(x.dtype), indices, scores


# speedup vs baseline: 1.6527x; 1.2087x over previous
import jax
import jax.numpy as jnp
from jax.experimental import pallas as pl

_TOPK = 8
_NUM_EXPERTS = 64
_BLOCK_ROWS = 512


def _gate_kernel(x_ref, e_ref, b_ref, w_ref, i_ref, s_ref):
    logits_t = jax.lax.dot_general(
        e_ref[...], x_ref[...],
        dimension_numbers=(((1,), (1,)), ((), ())),
        preferred_element_type=jnp.float32,
    )
    scores_t = jax.nn.sigmoid(logits_t) + b_ref[...]
    s_ref[...] = scores_t.T

    iota = jax.lax.broadcasted_iota(jnp.int32, scores_t.shape, 0)
    vals = scores_t
    neg_inf = jnp.float32(-jnp.inf)
    top_v = []
    top_i = []
    for _ in range(_TOPK):
        m = jnp.max(vals, axis=0, keepdims=True)
        idx = jnp.min(jnp.where(vals == m, iota, _NUM_EXPERTS),
                      axis=0, keepdims=True)
        top_v.append(m)
        top_i.append(idx)
        vals = jnp.where(iota == idx, neg_inf, vals)
    v = jnp.concatenate(top_v, axis=0)
    i_ref[...] = jnp.concatenate(top_i, axis=0).T
    w_ref[...] = (v / jnp.sum(v, axis=0, keepdims=True)).T


@jax.jit
def kernel(x, expert_embeddings, bias):
    n_rows, _ = x.shape
    n_exp = expert_embeddings.shape[0]
    grid = (n_rows // _BLOCK_ROWS,)
    bias2d = bias.reshape(n_exp, 1)
    weights, indices, scores = pl.pallas_call(
        _gate_kernel,
        grid=grid,
        in_specs=[
            pl.BlockSpec((_BLOCK_ROWS, x.shape[1]), lambda i: (i, 0)),
            pl.BlockSpec((n_exp, x.shape[1]), lambda i: (0, 0)),
            pl.BlockSpec((n_exp, 1), lambda i: (0, 0)),
        ],
        out_specs=[
            pl.BlockSpec((_BLOCK_ROWS, _TOPK), lambda i: (i, 0)),
            pl.BlockSpec((_BLOCK_ROWS, _TOPK), lambda i: (i, 0)),
            pl.BlockSpec((_BLOCK_ROWS, n_exp), lambda i: (i, 0)),
        ],
        out_shape=[
            jax.ShapeDtypeStruct((n_rows, _TOPK), jnp.float32),
            jax.ShapeDtypeStruct((n_rows, _TOPK), jnp.int32),
            jax.ShapeDtypeStruct((n_rows, n_exp), jnp.float32),
        ],
    )(x, expert_embeddings, bias2d)
    return weights.astype(x.dtype), indices, scores


# speedup vs baseline: 1.7672x; 1.0693x over previous
import jax
import jax.numpy as jnp
from jax.experimental import pallas as pl

_TOPK = 8
_NUM_EXPERTS = 64
_BLOCK_ROWS = 1024


def _gate_kernel(x_ref, e_ref, b_ref, w_ref, i_ref, s_ref):
    logits_t = jax.lax.dot_general(
        e_ref[...], x_ref[...],
        dimension_numbers=(((1,), (1,)), ((), ())),
        preferred_element_type=jnp.float32,
    )
    scores_t = jax.nn.sigmoid(logits_t) + b_ref[...]
    s_ref[...] = scores_t.T

    iota = jax.lax.broadcasted_iota(jnp.int32, scores_t.shape, 0)
    vals = scores_t
    neg_inf = jnp.float32(-jnp.inf)
    top_v = []
    top_i = []
    for _ in range(_TOPK):
        m = jnp.max(vals, axis=0, keepdims=True)
        idx = jnp.min(jnp.where(vals == m, iota, _NUM_EXPERTS),
                      axis=0, keepdims=True)
        top_v.append(m)
        top_i.append(idx)
        vals = jnp.where(iota == idx, neg_inf, vals)
    v = jnp.concatenate(top_v, axis=0)
    i_ref[...] = jnp.concatenate(top_i, axis=0).T
    w_ref[...] = (v / jnp.sum(v, axis=0, keepdims=True)).T


@jax.jit
def kernel(x, expert_embeddings, bias):
    n_rows, _ = x.shape
    n_exp = expert_embeddings.shape[0]
    grid = (n_rows // _BLOCK_ROWS,)
    bias2d = bias.reshape(n_exp, 1)
    weights, indices, scores = pl.pallas_call(
        _gate_kernel,
        grid=grid,
        in_specs=[
            pl.BlockSpec((_BLOCK_ROWS, x.shape[1]), lambda i: (i, 0)),
            pl.BlockSpec((n_exp, x.shape[1]), lambda i: (0, 0)),
            pl.BlockSpec((n_exp, 1), lambda i: (0, 0)),
        ],
        out_specs=[
            pl.BlockSpec((_BLOCK_ROWS, _TOPK), lambda i: (i, 0)),
            pl.BlockSpec((_BLOCK_ROWS, _TOPK), lambda i: (i, 0)),
            pl.BlockSpec((_BLOCK_ROWS, n_exp), lambda i: (i, 0)),
        ],
        out_shape=[
            jax.ShapeDtypeStruct((n_rows, _TOPK), jnp.float32),
            jax.ShapeDtypeStruct((n_rows, _TOPK), jnp.int32),
            jax.ShapeDtypeStruct((n_rows, n_exp), jnp.float32),
        ],
    )(x, expert_embeddings, bias2d)
    return weights.astype(x.dtype), indices, scores
